# baseline (speedup 1.0000x reference)
import functools

import jax
import jax.numpy as jnp
from jax import lax
from jax.experimental import pallas as pl
from jax.experimental.pallas import tpu as pltpu
from jax.experimental.pallas import tpu_sc as plsc

_VOCAB = 49408
_MAXPOS = 77
_D = 768
_B = 4096
_L = 77
_N = _B * _L
_NC = 2
_NS = 16
_NW = _NC * _NS
_PER_W = _N // _NW
_K = 16
_NCHUNK = _PER_W // _K
_PADPOS = _MAXPOS + _K - 1
_LANES = 16


def _sc_body(table_hbm, ids_hbm, pospad_hbm, out_hbm, idx_v, pospad_v, buf_v,
             sem_g):
    wid = lax.axis_index("s") * _NC + lax.axis_index("c")
    base = wid * _PER_W
    pltpu.sync_copy(ids_hbm.at[pl.ds(base, _PER_W)], idx_v)
    pltpu.sync_copy(pospad_hbm, pospad_v)

    def chunk(c, carry):
        row0 = c * _K
        p0 = lax.rem(row0, _MAXPOS)
        pltpu.async_copy(
            table_hbm.at[idx_v.at[pl.ds(row0, _K)]], buf_v, sem_g
        ).wait()

        def add_row(i, carry2):
            for k in range(0, _D, _LANES):
                v = buf_v[i, pl.ds(k, _LANES)] + pospad_v[p0 + i, pl.ds(k, _LANES)]
                buf_v[i, pl.ds(k, _LANES)] = v
            return carry2

        lax.fori_loop(0, _K, add_row, 0)
        pltpu.sync_copy(buf_v, out_hbm.at[pl.ds(base + row0, _K)])
        return carry

    lax.fori_loop(0, _NCHUNK, chunk, 0)


def kernel(input_ids, token_table, pos_table):
    ids_flat = input_ids.reshape(_N)
    pos_padded = jnp.concatenate([pos_table, pos_table[: _K - 1]], axis=0)

    mesh = plsc.VectorSubcoreMesh(core_axis_name="c", subcore_axis_name="s")
    run = pl.kernel(
        _sc_body,
        mesh=mesh,
        out_type=jax.ShapeDtypeStruct((_N, _D), jnp.float32),
        scratch_types=[
            pltpu.VMEM((_PER_W,), jnp.int32),
            pltpu.VMEM((_PADPOS, _D), jnp.float32),
            pltpu.VMEM((_K, _D), jnp.float32),
            pltpu.SemaphoreType.DMA,
        ],
    )
    out = run(token_table, ids_flat, pos_padded)
    return out.reshape(_B, _L, _D)


# speedup vs baseline: 1.4126x; 1.4126x over previous
import jax
import jax.numpy as jnp
from jax import lax
from jax.experimental import pallas as pl
from jax.experimental.pallas import tpu as pltpu
from jax.experimental.pallas import tpu_sc as plsc

_MAXPOS = 77
_D = 768
_B = 4096
_L = 77
_N = _B * _L
_NC = 2
_NS = 16
_NW = _NC * _NS
_PER_W = _N // _NW
_K = 8
_NBUF = 4
_NCHUNK = _PER_W // _K
_NGROUP = _NCHUNK // _NBUF
_PADPOS = _MAXPOS + _K - 1
_LANES = 16


def _sc_body(table_hbm, ids_hbm, pospad_hbm, out_hbm, idx_v, pospad_v, buf_v,
             sem0, sem1, sem2, sem3):
    sems = [sem0, sem1, sem2, sem3]
    wid = lax.axis_index("s") * _NC + lax.axis_index("c")
    base = wid * _PER_W
    pltpu.sync_copy(ids_hbm.at[pl.ds(base, _PER_W)], idx_v)
    pltpu.sync_copy(pospad_hbm, pospad_v)

    def start_gather(c, b):
        pltpu.async_copy(
            table_hbm.at[idx_v.at[pl.ds(c * _K, _K)]], buf_v.at[b], sems[b]
        )

    def wait_gather(b):
        pltpu.make_async_copy(
            table_hbm.at[idx_v.at[pl.ds(0, _K)]], buf_v.at[b], sems[b]
        ).wait()

    for b in range(_NBUF - 1):
        start_gather(b, b)

    def group(gi, carry):
        for b in range(_NBUF):
            c = gi * _NBUF + b
            bpre = (b + _NBUF - 1) % _NBUF

            @pl.when(c + _NBUF - 1 < _NCHUNK)
            def _():
                start_gather(c + _NBUF - 1, bpre)

            wait_gather(b)
            p0 = lax.rem(c * _K, _MAXPOS)

            def add_row(i, carry2):
                for k in range(0, _D, _LANES):
                    plsc.addupdate(
                        buf_v.at[b, i, pl.ds(k, _LANES)],
                        pospad_v[p0 + i, pl.ds(k, _LANES)],
                    )
                return carry2

            lax.fori_loop(0, _K, add_row, 0)
            pltpu.sync_copy(buf_v.at[b], out_hbm.at[pl.ds(base + c * _K, _K)])
        return carry

    lax.fori_loop(0, _NGROUP, group, 0)


def kernel(input_ids, token_table, pos_table):
    ids_flat = input_ids.reshape(_N)
    pos_padded = jnp.concatenate([pos_table, pos_table[: _K - 1]], axis=0)

    mesh = plsc.VectorSubcoreMesh(core_axis_name="c", subcore_axis_name="s")
    run = pl.kernel(
        _sc_body,
        mesh=mesh,
        out_type=jax.ShapeDtypeStruct((_N, _D), jnp.float32),
        scratch_types=[
            pltpu.VMEM((_PER_W,), jnp.int32),
            pltpu.VMEM((_PADPOS, _D), jnp.float32),
            pltpu.VMEM((_NBUF, _K, _D), jnp.float32),
            pltpu.SemaphoreType.DMA,
            pltpu.SemaphoreType.DMA,
            pltpu.SemaphoreType.DMA,
            pltpu.SemaphoreType.DMA,
        ],
    )
    out = run(token_table, ids_flat, pos_padded)
    return out.reshape(_B, _L, _D)


# speedup vs baseline: 1.9772x; 1.3997x over previous
import jax
import jax.numpy as jnp
from jax import lax
from jax.experimental import pallas as pl
from jax.experimental.pallas import tpu as pltpu
from jax.experimental.pallas import tpu_sc as plsc

_MAXPOS = 77
_D = 768
_B = 4096
_L = 77
_N = _B * _L
_NC = 2
_NS = 16
_NW = _NC * _NS
_PER_W = _N // _NW
_K = 16
_NBUF = 4
_NCHUNK = _PER_W // _K
_NGROUP = _NCHUNK // _NBUF
_PADPOS = _MAXPOS
_LANES = 16


def _sc_body(table_hbm, ids_hbm, pospad_hbm, out_hbm, idx_v, pospad_v, buf_v,
             sg0, sg1, sg2, sg3, ss0, ss1, ss2, ss3):
    sg = [sg0, sg1, sg2, sg3]
    ss = [ss0, ss1, ss2, ss3]
    wid = lax.axis_index("s") * _NC + lax.axis_index("c")
    base = wid * _PER_W
    pltpu.sync_copy(ids_hbm.at[pl.ds(base, _PER_W)], idx_v)
    pltpu.sync_copy(pospad_hbm, pospad_v)

    def start_gather(c, b):
        pltpu.async_copy(
            table_hbm.at[idx_v.at[pl.ds(c * _K, _K)]], buf_v.at[b], sg[b]
        )

    def wait_gather(b):
        pltpu.make_async_copy(
            table_hbm.at[idx_v.at[pl.ds(0, _K)]], buf_v.at[b], sg[b]
        ).wait()

    def start_store(c, b):
        pltpu.async_copy(buf_v.at[b], out_hbm.at[pl.ds(base + c * _K, _K)],
                         ss[b])

    def wait_store(c, b):
        pltpu.make_async_copy(
            buf_v.at[b], out_hbm.at[pl.ds(base + c * _K, _K)], ss[b]
        ).wait()

    for b in range(_NBUF - 2):
        start_gather(b, b)

    def group(gi, carry):
        for b in range(_NBUF):
            c = gi * _NBUF + b
            b2 = (b + 2) % _NBUF

            @pl.when(c >= 2)
            def _():
                wait_store(c - 2, b2)

            @pl.when(c + 2 < _NCHUNK)
            def _():
                start_gather(c + 2, b2)

            wait_gather(b)
            p0 = lax.rem(c * _K, _MAXPOS)

            @plsc.parallel_loop(0, _K, unroll=2)
            def _(i):
                p = p0 + i
                p = jnp.where(p >= _MAXPOS, p - _MAXPOS, p)
                for k in range(0, _D, _LANES):
                    plsc.addupdate(
                        buf_v.at[b, i, pl.ds(k, _LANES)],
                        pospad_v[p, pl.ds(k, _LANES)],
                    )

            start_store(c, b)
        return carry

    lax.fori_loop(0, _NGROUP, group, 0)
    wait_store(_NCHUNK - 2, (_NCHUNK - 2) % _NBUF)
    wait_store(_NCHUNK - 1, (_NCHUNK - 1) % _NBUF)


def kernel(input_ids, token_table, pos_table):
    ids_flat = input_ids.reshape(_N)

    mesh = plsc.VectorSubcoreMesh(core_axis_name="c", subcore_axis_name="s")
    run = pl.kernel(
        _sc_body,
        mesh=mesh,
        out_type=jax.ShapeDtypeStruct((_N, _D), jnp.float32),
        scratch_types=[
            pltpu.VMEM((_PER_W,), jnp.int32),
            pltpu.VMEM((_PADPOS, _D), jnp.float32),
            pltpu.VMEM((_NBUF, _K, _D), jnp.float32),
            pltpu.SemaphoreType.DMA,
            pltpu.SemaphoreType.DMA,
            pltpu.SemaphoreType.DMA,
            pltpu.SemaphoreType.DMA,
            pltpu.SemaphoreType.DMA,
            pltpu.SemaphoreType.DMA,
            pltpu.SemaphoreType.DMA,
            pltpu.SemaphoreType.DMA,
        ],
    )
    out = run(token_table, ids_flat, pos_table)
    return out.reshape(_B, _L, _D)
